# baseline (speedup 1.0000x reference)
import functools

import jax
import jax.numpy as jnp
from jax import lax
from jax.experimental import pallas as pl
from jax.experimental.pallas import tpu as pltpu
from jax.experimental.pallas import tpu_sc as plsc

T = 2048
D_MODEL = 1024
EXPERT_DIM = 2048
NUM_EXPERTS = 8
K = 2
ENT = T * K
BT = 256
G_MAX = ENT // BT + NUM_EXPERTS
N_PAD = G_MAX * BT
NW = 32
EPW = ENT // NW
NCH = EPW // 16
NCH_ALL = ENT // 16


def _gating_body(x_ref, gw_ref, gb_ref, eid_ref, wts_ref):
    x = x_ref[...]
    logits = jnp.dot(x, gw_ref[...], preferred_element_type=jnp.float32)
    logits = logits + gb_ref[...]
    iota = lax.broadcasted_iota(jnp.int32, (T, NUM_EXPERTS), 1)
    m1 = jnp.max(logits, axis=-1, keepdims=True)
    idx1 = jnp.min(jnp.where(logits == m1, iota, NUM_EXPERTS), axis=-1,
                   keepdims=True)
    masked = jnp.where(iota == idx1, -jnp.inf, logits)
    m2 = jnp.max(masked, axis=-1, keepdims=True)
    idx2 = jnp.min(jnp.where(masked == m2, iota, NUM_EXPERTS), axis=-1,
                   keepdims=True)
    e2 = jnp.exp(m2 - m1)
    s = 1.0 + e2
    eid_ref[...] = jnp.concatenate([idx1, idx2], axis=1)
    wts_ref[...] = jnp.concatenate([1.0 / s, e2 / s], axis=1)


def _gating(x_flat, gate_w, gb):
    return pl.pallas_call(
        _gating_body,
        grid=(1,),
        in_specs=[
            pl.BlockSpec((T, D_MODEL), lambda i: (0, 0)),
            pl.BlockSpec((D_MODEL, NUM_EXPERTS), lambda i: (0, 0)),
            pl.BlockSpec((1, NUM_EXPERTS), lambda i: (0, 0)),
        ],
        out_specs=[
            pl.BlockSpec((T, K), lambda i: (0, 0)),
            pl.BlockSpec((T, K), lambda i: (0, 0)),
        ],
        out_shape=[
            jax.ShapeDtypeStruct((T, K), jnp.int32),
            jax.ShapeDtypeStruct((T, K), jnp.float32),
        ],
    )(x_flat, gate_w, gb)


def _make_route(TH):
    ENT_H = TH * K
    NCH_H = ENT_H // NW // 16
    NCH_ALL_H = ENT_H // 16
    G_MAX_H = ENT_H // BT + NUM_EXPERTS
    N_PAD_H = G_MAX_H * BT
    RH = NCH_H // 2
    NPF = min(4, NCH_H)

    def _route_body(eid_hbm, tok_hbm, x_hbm, xs_hbm, dest_hbm, meta_hbm,
                    eid_all_v, rank_all_v, tok_v, dest_v, dsm_v, base_v,
                    rows_v, sem, sem2):
        wid = lax.axis_index("s") * 2 + lax.axis_index("c")
        pltpu.sync_copy(eid_hbm, eid_all_v)
        pltpu.sync_copy(tok_hbm.at[wid], tok_v)

        gat = [pltpu.async_copy(x_hbm.at[tok_v.at[cc]], rows_v.at[cc % 4],
                                sem)
               for cc in range(NPF)]

        ones16 = jnp.ones((16,), jnp.int32)

        def scan_body(c, carries):
            eid16 = eid_all_v[c]
            rank16 = jnp.zeros((16,), jnp.int32)
            new = []
            for e in range(NUM_EXPERTS):
                m = eid16 == jnp.full((16,), e, jnp.int32)
                mi = jnp.where(m, ones16, ones16 - ones16)
                pc = plsc.cumsum(mi)
                ce = jnp.full((16,), carries[e], jnp.int32)
                rank16 = jnp.where(m, ce + pc - ones16, rank16)
                new.append(carries[e] + jnp.sum(mi))
            rank_all_v[c] = rank16
            return tuple(new)

        cnt = lax.fori_loop(0, NCH_ALL_H, scan_body,
                            tuple(jnp.int32(0) for _ in range(NUM_EXPERTS)))

        iota16 = lax.iota(jnp.int32, 16)
        cnt_v = jnp.zeros((16,), jnp.int32)
        for e in range(NUM_EXPERTS):
            cnt_v = jnp.where(iota16 == jnp.full((16,), e, jnp.int32),
                              jnp.full((16,), cnt[e], jnp.int32), cnt_v)
        p_v = ((cnt_v + (BT - 1)) // BT) * BT
        cum_v = plsc.cumsum(p_v)
        starts_v = cum_v - p_v
        base_v[...] = starts_v

        @pl.when(wid == 0)
        def _meta():
            dest_v[0] = cum_v // BT
            pltpu.sync_copy(dest_v.at[0], meta_hbm)

        for cc in range(NCH_H):
            cg = wid * NCH_H + cc
            eid16 = eid_all_v[cg]
            rank16 = rank_all_v[cg]
            dest16 = plsc.load_gather(base_v, [eid16]) + rank16
            dest_v[cc] = dest16

        for r in range(RH):
            row0 = jnp.full((16,), 2 * r, jnp.int32) + iota16 // 8
            col0 = (2 * iota16) % jnp.full((16,), 16, jnp.int32)
            dsm_v[r] = plsc.load_gather(dest_v, [row0, col0])
            col1 = (2 * iota16 + ones16) % jnp.full((16,), 16, jnp.int32)
            dsm_v[RH + r] = plsc.load_gather(dest_v, [row0, col1])
        pltpu.sync_copy(dsm_v.at[pl.ds(0, RH)],
                        dest_hbm.at[pl.ds(RH * wid, RH)])
        pltpu.sync_copy(dsm_v.at[pl.ds(RH, RH)],
                        dest_hbm.at[pl.ds(TH // 16 + RH * wid, RH)])

        scat = []
        for cc in range(NCH_H):
            b = cc % 4
            gat[cc].wait()
            scat.append(
                pltpu.async_copy(rows_v.at[b], xs_hbm.at[dest_v.at[cc]],
                                 sem2))
            if cc + NPF < NCH_H:
                scat[cc].wait()
                gat.append(pltpu.async_copy(x_hbm.at[tok_v.at[cc + NPF]],
                                            rows_v.at[b], sem))
        for cc in range(max(0, NCH_H - NPF), NCH_H):
            scat[cc].wait()

    def _route(eid3, tok3, x_flat):
        mesh = plsc.VectorSubcoreMesh(core_axis_name="c",
                                      subcore_axis_name="s")
        f = pl.kernel(
            _route_body,
            out_type=[
                jax.ShapeDtypeStruct((N_PAD_H, D_MODEL), jnp.float32),
                jax.ShapeDtypeStruct((NCH_ALL_H, 16), jnp.int32),
                jax.ShapeDtypeStruct((16,), jnp.int32),
            ],
            mesh=mesh,
            scratch_types=[
                pltpu.VMEM((NCH_ALL_H, 16), jnp.int32),
                pltpu.VMEM((NCH_ALL_H, 16), jnp.int32),
                pltpu.VMEM((NCH_H, 16), jnp.int32),
                pltpu.VMEM((NCH_H, 16), jnp.int32),
                pltpu.VMEM((NCH_H, 16), jnp.int32),
                pltpu.VMEM((16,), jnp.int32),
                pltpu.VMEM((4, 16, D_MODEL), jnp.float32),
                pltpu.SemaphoreType.DMA,
                pltpu.SemaphoreType.DMA,
            ],
            compiler_params=pltpu.CompilerParams(needs_layout_passes=False),
        )
        return f(eid3, tok3, x_flat)

    return _route


def _ffn_body(meta_ref, xs_ref, w1_ref, b1_ref, w2_ref, b2_ref, out_ref):
    g = pl.program_id(0)

    @pl.when(g < meta_ref[8])
    def _compute():
        x = xs_ref[...]
        h = jnp.dot(x, w1_ref[0], preferred_element_type=jnp.float32)
        h = jnp.maximum(h + b1_ref[0], 0.0)
        o = jnp.dot(h, w2_ref[0], preferred_element_type=jnp.float32)
        out_ref[...] = o + b2_ref[0]


def _expert_of(g, meta_ref):
    e = jnp.int32(0)
    for i in range(NUM_EXPERTS):
        e = e + (g >= meta_ref[i]).astype(jnp.int32)
    return jnp.minimum(e, NUM_EXPERTS - 1)


def _make_ffn(TH):
    G_MAX_H = TH * K // BT + NUM_EXPERTS
    N_PAD_H = G_MAX_H * BT

    def _ffn(meta, xs, W1, b1r, W2, b2r):
        grid_spec = pltpu.PrefetchScalarGridSpec(
            num_scalar_prefetch=1,
            grid=(G_MAX_H,),
            in_specs=[
                pl.BlockSpec((BT, D_MODEL), lambda g, m: (g, 0)),
                pl.BlockSpec((1, D_MODEL, EXPERT_DIM),
                             lambda g, m: (_expert_of(g, m), 0, 0)),
                pl.BlockSpec((1, 1, EXPERT_DIM),
                             lambda g, m: (_expert_of(g, m), 0, 0)),
                pl.BlockSpec((1, EXPERT_DIM, D_MODEL),
                             lambda g, m: (_expert_of(g, m), 0, 0)),
                pl.BlockSpec((1, 1, D_MODEL),
                             lambda g, m: (_expert_of(g, m), 0, 0)),
            ],
            out_specs=pl.BlockSpec((BT, D_MODEL), lambda g, m: (g, 0)),
        )
        return pl.pallas_call(
            _ffn_body,
            grid_spec=grid_spec,
            out_shape=jax.ShapeDtypeStruct((N_PAD_H, D_MODEL), jnp.float32),
            compiler_params=pltpu.CompilerParams(
                vmem_limit_bytes=120 * 1024 * 1024),
        )(meta, xs, W1, b1r, W2, b2r)

    return _ffn


def _make_cgather(TH):
    ENT_H = TH * K
    NCH_H = ENT_H // NW // 16
    G_MAX_H = ENT_H // BT + NUM_EXPERTS
    N_PAD_H = G_MAX_H * BT

    def _cgather_body(ys_hbm, dest_hbm, yi_hbm, dest_v, rows_v, sem, sem2):
        wid = lax.axis_index("s") * 2 + lax.axis_index("c")
        pltpu.sync_copy(dest_hbm.at[wid], dest_v)
        stor = []
        for cc in range(NCH_H):
            b = cc % 2
            if cc >= 2:
                stor[cc - 2].wait()
            pltpu.async_copy(ys_hbm.at[dest_v.at[cc]], rows_v.at[b],
                             sem).wait()
            base = (wid * NCH_H + cc) * 16
            pltpu.sync_copy(rows_v.at[b], yi_hbm.at[pl.ds(base, 16)])

    def _cgather_body2(ys_hbm, dest_hbm, yi_hbm, dest_v, rows_v, sem, sem2):
        wid = lax.axis_index("s") * 2 + lax.axis_index("c")
        pltpu.sync_copy(dest_hbm.at[wid], dest_v)
        stor = []
        for cc in range(NCH_H):
            b = cc % 2
            if cc >= 2:
                stor[cc - 2].wait()
            pltpu.async_copy(ys_hbm.at[dest_v.at[cc]], rows_v.at[b],
                             sem).wait()
            base = (wid * NCH_H + cc) * 16
            stor.append(
                pltpu.async_copy(rows_v.at[b], yi_hbm.at[pl.ds(base, 16)],
                                 sem2))
        for cc in range(max(0, NCH_H - 2), NCH_H):
            stor[cc].wait()

    def _cgather(ys, dest3):
        mesh = plsc.VectorSubcoreMesh(core_axis_name="c",
                                      subcore_axis_name="s")
        f = pl.kernel(
            _cgather_body2,
            out_type=jax.ShapeDtypeStruct((ENT_H, D_MODEL), jnp.float32),
            mesh=mesh,
            scratch_types=[
                pltpu.VMEM((NCH_H, 16), jnp.int32),
                pltpu.VMEM((2, 16, D_MODEL), jnp.float32),
                pltpu.SemaphoreType.DMA,
                pltpu.SemaphoreType.DMA,
            ],
            compiler_params=pltpu.CompilerParams(needs_layout_passes=False),
        )
        return f(ys, dest3)

    return _cgather


def _combine_body(y0_ref, y1_ref, w_ref, out_ref):
    w = w_ref[...]
    out_ref[...] = w[:, :1] * y0_ref[...] + w[:, 1:] * y1_ref[...]


_BTD = 512


def _make_combine(TH):
    nb = TH // _BTD

    def _combine(ysm, wts):
        return pl.pallas_call(
            _combine_body,
            grid=(nb,),
            in_specs=[
                pl.BlockSpec((_BTD, D_MODEL), lambda t: (t, 0)),
                pl.BlockSpec((_BTD, D_MODEL), lambda t: (t + nb, 0)),
                pl.BlockSpec((_BTD, K), lambda t: (t, 0)),
            ],
            out_specs=pl.BlockSpec((_BTD, D_MODEL), lambda t: (t, 0)),
            out_shape=jax.ShapeDtypeStruct((TH, D_MODEL), jnp.float32),
        )(ysm, ysm, wts)

    return _combine


TH = T // 2
_route_h = _make_route(TH)
_ffn_h = _make_ffn(TH)
_cgather_h = _make_cgather(TH)
_combine_h = _make_combine(TH)


def kernel(x, W1, b1, W2, b2, gate_w, gate_b, bias):
    seq_len, batch_size, d_model = x.shape
    x_flat = x.reshape(-1, d_model)
    gb = (gate_b + bias).reshape(1, NUM_EXPERTS)
    b1r = b1.reshape(NUM_EXPERTS, 1, EXPERT_DIM)
    b2r = b2.reshape(NUM_EXPERTS, 1, D_MODEL)

    eid, wts = _gating(x_flat, gate_w, gb)

    outs = []
    for h in range(2):
        t0 = h * TH
        eid3 = eid[t0:t0 + TH].reshape(TH * K // 16, 16)
        tok3 = ((jnp.arange(TH * K, dtype=jnp.int32) // K)
                + jnp.int32(t0)).reshape(NW, TH * K // NW // 16, 16)
        xs, dsm, meta = _route_h(eid3, tok3, x_flat)
        dest3 = dsm.reshape(NW, TH * K // NW // 16, 16)
        ys = _ffn_h(meta, xs, W1, b1r, W2, b2r)
        ysm = _cgather_h(ys, dest3)
        outs.append(_combine_h(ysm, wts[t0:t0 + TH]))

    out = jnp.concatenate(outs, axis=0)
    return out.reshape(seq_len, batch_size, d_model)


# speedup vs baseline: 1.2143x; 1.2143x over previous
import functools

import jax
import jax.numpy as jnp
from jax import lax
from jax.experimental import pallas as pl
from jax.experimental.pallas import tpu as pltpu
from jax.experimental.pallas import tpu_sc as plsc

T = 2048
D_MODEL = 1024
EXPERT_DIM = 2048
NUM_EXPERTS = 8
K = 2
ENT = T * K
BT = 256
G_MAX = ENT // BT + NUM_EXPERTS
N_PAD = G_MAX * BT
NW = 32
EPW = ENT // NW
NCH = EPW // 16
NCH_ALL = ENT // 16


def _gating_body(x_ref, gw_ref, gb_ref, eid_ref, wts_ref):
    x = x_ref[...]
    logits = jnp.dot(x, gw_ref[...], preferred_element_type=jnp.float32)
    logits = logits + gb_ref[...]
    iota = lax.broadcasted_iota(jnp.int32, (T, NUM_EXPERTS), 1)
    m1 = jnp.max(logits, axis=-1, keepdims=True)
    idx1 = jnp.min(jnp.where(logits == m1, iota, NUM_EXPERTS), axis=-1,
                   keepdims=True)
    masked = jnp.where(iota == idx1, -jnp.inf, logits)
    m2 = jnp.max(masked, axis=-1, keepdims=True)
    idx2 = jnp.min(jnp.where(masked == m2, iota, NUM_EXPERTS), axis=-1,
                   keepdims=True)
    e2 = jnp.exp(m2 - m1)
    s = 1.0 + e2
    eid_ref[...] = jnp.concatenate([idx1, idx2], axis=1)
    wts_ref[...] = jnp.concatenate([1.0 / s, e2 / s], axis=1)


def _gating(x_flat, gate_w, gb):
    return pl.pallas_call(
        _gating_body,
        grid=(1,),
        in_specs=[
            pl.BlockSpec((T, D_MODEL), lambda i: (0, 0)),
            pl.BlockSpec((D_MODEL, NUM_EXPERTS), lambda i: (0, 0)),
            pl.BlockSpec((1, NUM_EXPERTS), lambda i: (0, 0)),
        ],
        out_specs=[
            pl.BlockSpec((T, K), lambda i: (0, 0)),
            pl.BlockSpec((T, K), lambda i: (0, 0)),
        ],
        out_shape=[
            jax.ShapeDtypeStruct((T, K), jnp.int32),
            jax.ShapeDtypeStruct((T, K), jnp.float32),
        ],
    )(x_flat, gate_w, gb)


def _route_body(eid_hbm, tok_hbm, x_hbm, xs_hbm, dest_hbm, meta_hbm,
                eid_all_v, rank_all_v, tok_v, dest_v, dsm_v, base_v, rows_v,
                sem, sem2):
    wid = lax.axis_index("s") * 2 + lax.axis_index("c")
    pltpu.sync_copy(eid_hbm, eid_all_v)
    pltpu.sync_copy(tok_hbm.at[wid], tok_v)

    gat = [pltpu.async_copy(x_hbm.at[tok_v.at[cc]], rows_v.at[cc % 3], sem)
           for cc in range(3)]

    ones16 = jnp.ones((16,), jnp.int32)

    def scan_body(c, carries):
        eid16 = eid_all_v[c]
        rank16 = jnp.zeros((16,), jnp.int32)
        new = []
        for e in range(NUM_EXPERTS):
            m = eid16 == jnp.full((16,), e, jnp.int32)
            mi = jnp.where(m, ones16, ones16 - ones16)
            pc = plsc.cumsum(mi)
            ce = jnp.full((16,), carries[e], jnp.int32)
            rank16 = jnp.where(m, ce + pc - ones16, rank16)
            new.append(carries[e] + jnp.sum(mi))
        rank_all_v[c] = rank16
        return tuple(new)

    cnt = lax.fori_loop(0, NCH_ALL, scan_body,
                        tuple(jnp.int32(0) for _ in range(NUM_EXPERTS)))

    iota16 = lax.iota(jnp.int32, 16)
    cnt_v = jnp.zeros((16,), jnp.int32)
    for e in range(NUM_EXPERTS):
        cnt_v = jnp.where(iota16 == jnp.full((16,), e, jnp.int32),
                          jnp.full((16,), cnt[e], jnp.int32), cnt_v)
    p_v = ((cnt_v + (BT - 1)) // BT) * BT
    cum_v = plsc.cumsum(p_v)
    starts_v = cum_v - p_v
    base_v[...] = starts_v

    @pl.when(wid == 0)
    def _meta():
        dest_v[0] = cum_v // BT
        pltpu.sync_copy(dest_v.at[0], meta_hbm)

    for cc in range(NCH):
        cg = wid * NCH + cc
        eid16 = eid_all_v[cg]
        rank16 = rank_all_v[cg]
        dest16 = plsc.load_gather(base_v, [eid16]) + rank16
        dest_v[cc] = dest16

    for r in range(4):
        row0 = jnp.full((16,), 2 * r, jnp.int32) + iota16 // 8
        col0 = (2 * iota16) % jnp.full((16,), 16, jnp.int32)
        dsm_v[r] = plsc.load_gather(dest_v, [row0, col0])
        col1 = (2 * iota16 + ones16) % jnp.full((16,), 16, jnp.int32)
        dsm_v[4 + r] = plsc.load_gather(dest_v, [row0, col1])
    pltpu.sync_copy(dsm_v.at[pl.ds(0, 4)],
                    dest_hbm.at[pl.ds(4 * wid, 4)])
    pltpu.sync_copy(dsm_v.at[pl.ds(4, 4)],
                    dest_hbm.at[pl.ds(T // 16 + 4 * wid, 4)])

    scat = []
    for cc in range(NCH):
        b = cc % 3
        gat[cc].wait()
        scat.append(
            pltpu.async_copy(rows_v.at[b], xs_hbm.at[dest_v.at[cc]], sem2))
        if cc + 3 < NCH:
            scat[cc].wait()
            gat.append(pltpu.async_copy(x_hbm.at[tok_v.at[cc + 3]],
                                        rows_v.at[b], sem))
    for cc in range(max(0, NCH - 3), NCH):
        scat[cc].wait()


def _route(eid3, tok3, x_flat):
    mesh = plsc.VectorSubcoreMesh(core_axis_name="c", subcore_axis_name="s")
    f = pl.kernel(
        _route_body,
        out_type=[
            jax.ShapeDtypeStruct((N_PAD, D_MODEL), jnp.float32),
            jax.ShapeDtypeStruct((NCH_ALL, 16), jnp.int32),
            jax.ShapeDtypeStruct((16,), jnp.int32),
        ],
        mesh=mesh,
        scratch_types=[
            pltpu.VMEM((NCH_ALL, 16), jnp.int32),
            pltpu.VMEM((NCH_ALL, 16), jnp.int32),
            pltpu.VMEM((NCH, 16), jnp.int32),
            pltpu.VMEM((NCH, 16), jnp.int32),
            pltpu.VMEM((NCH, 16), jnp.int32),
            pltpu.VMEM((16,), jnp.int32),
            pltpu.VMEM((3, 16, D_MODEL), jnp.float32),
            pltpu.SemaphoreType.DMA,
            pltpu.SemaphoreType.DMA,
        ],
        compiler_params=pltpu.CompilerParams(needs_layout_passes=False),
    )
    return f(eid3, tok3, x_flat)


def _ffn_body(meta_ref, xs_ref, w1_ref, b1_ref, w2_ref, b2_ref, out_ref):
    g = pl.program_id(0)

    @pl.when(g < meta_ref[8])
    def _compute():
        x = xs_ref[...]
        h = jnp.dot(x, w1_ref[0], preferred_element_type=jnp.float32)
        h = jnp.maximum(h + b1_ref[0], 0.0)
        o = jnp.dot(h, w2_ref[0], preferred_element_type=jnp.float32)
        out_ref[...] = o + b2_ref[0]


def _expert_of(g, meta_ref):
    e = jnp.int32(0)
    for i in range(NUM_EXPERTS):
        e = e + (g >= meta_ref[i]).astype(jnp.int32)
    return jnp.minimum(e, NUM_EXPERTS - 1)


def _ffn(meta, xs, W1, b1r, W2, b2r):
    grid_spec = pltpu.PrefetchScalarGridSpec(
        num_scalar_prefetch=1,
        grid=(G_MAX,),
        in_specs=[
            pl.BlockSpec((BT, D_MODEL), lambda g, m: (g, 0)),
            pl.BlockSpec((1, D_MODEL, EXPERT_DIM),
                         lambda g, m: (_expert_of(g, m), 0, 0)),
            pl.BlockSpec((1, 1, EXPERT_DIM),
                         lambda g, m: (_expert_of(g, m), 0, 0)),
            pl.BlockSpec((1, EXPERT_DIM, D_MODEL),
                         lambda g, m: (_expert_of(g, m), 0, 0)),
            pl.BlockSpec((1, 1, D_MODEL),
                         lambda g, m: (_expert_of(g, m), 0, 0)),
        ],
        out_specs=pl.BlockSpec((BT, D_MODEL), lambda g, m: (g, 0)),
    )
    return pl.pallas_call(
        _ffn_body,
        grid_spec=grid_spec,
        out_shape=jax.ShapeDtypeStruct((N_PAD, D_MODEL), jnp.float32),
        compiler_params=pltpu.CompilerParams(
            vmem_limit_bytes=120 * 1024 * 1024),
    )(meta, xs, W1, b1r, W2, b2r)


def _cgather_body(ys_hbm, dest_hbm, yi_hbm, dest_v, rows_v, sem, sem2):
    wid = lax.axis_index("s") * 2 + lax.axis_index("c")
    pltpu.sync_copy(dest_hbm.at[wid], dest_v)
    stor = []
    for cc in range(NCH):
        b = cc % 2
        if cc >= 2:
            stor[cc - 2].wait()
        pltpu.async_copy(ys_hbm.at[dest_v.at[cc]], rows_v.at[b], sem).wait()
        base = (wid * NCH + cc) * 16
        stor.append(
            pltpu.async_copy(rows_v.at[b], yi_hbm.at[pl.ds(base, 16)], sem2))
    stor[NCH - 2].wait()
    stor[NCH - 1].wait()


def _cgather(ys, dest3):
    mesh = plsc.VectorSubcoreMesh(core_axis_name="c", subcore_axis_name="s")
    f = pl.kernel(
        _cgather_body,
        out_type=jax.ShapeDtypeStruct((ENT, D_MODEL), jnp.float32),
        mesh=mesh,
        scratch_types=[
            pltpu.VMEM((NCH, 16), jnp.int32),
            pltpu.VMEM((2, 16, D_MODEL), jnp.float32),
            pltpu.SemaphoreType.DMA,
            pltpu.SemaphoreType.DMA,
        ],
        compiler_params=pltpu.CompilerParams(needs_layout_passes=False),
    )
    return f(ys, dest3)


def _combine_body(y0_ref, y1_ref, w_ref, out_ref):
    w = w_ref[...]
    out_ref[...] = w[:, :1] * y0_ref[...] + w[:, 1:] * y1_ref[...]


_BTD = 512


def _combine(ysm, wts):
    return pl.pallas_call(
        _combine_body,
        grid=(T // _BTD,),
        in_specs=[
            pl.BlockSpec((_BTD, D_MODEL), lambda t: (t, 0)),
            pl.BlockSpec((_BTD, D_MODEL), lambda t: (t + T // _BTD, 0)),
            pl.BlockSpec((_BTD, K), lambda t: (t, 0)),
        ],
        out_specs=pl.BlockSpec((_BTD, D_MODEL), lambda t: (t, 0)),
        out_shape=jax.ShapeDtypeStruct((T, D_MODEL), jnp.float32),
    )(ysm, ysm, wts)


def kernel(x, W1, b1, W2, b2, gate_w, gate_b, bias):
    seq_len, batch_size, d_model = x.shape
    x_flat = x.reshape(-1, d_model)
    gb = (gate_b + bias).reshape(1, NUM_EXPERTS)

    eid, wts = _gating(x_flat, gate_w, gb)
    eid3 = eid.reshape(NCH_ALL, 16)
    tok3 = (jnp.arange(ENT, dtype=jnp.int32) // K).reshape(NW, NCH, 16)

    xs, dsm, meta = _route(eid3, tok3, x_flat)
    dest3 = dsm.reshape(NW, NCH, 16)
    ys = _ffn(meta, xs, W1, b1.reshape(NUM_EXPERTS, 1, EXPERT_DIM),
              W2, b2.reshape(NUM_EXPERTS, 1, D_MODEL))
    ysm = _cgather(ys, dest3)
    out = _combine(ysm, wts)
    return out.reshape(seq_len, batch_size, d_model)
